# baseline (speedup 1.0000x reference)
import functools
import math

import jax
import jax.numpy as jnp
from jax.experimental import pallas as pl
from jax.experimental.pallas import tpu as pltpu

_SQRT_2_OVER_PI = math.sqrt(2.0 / math.pi)

_CHUNK = 128
_NBUF = 16


def _gelu(x):
    inner = _SQRT_2_OVER_PI * (x + 0.044715 * (x * x * x))
    return 0.5 * x * (1.0 + jnp.tanh(inner))


def _body(x_hbm, o_hbm, xbuf, ybuf, in_sem, out_sem, *, n_chunks):
    def copy_in(i, slot):
        return pltpu.make_async_copy(
            x_hbm.at[pl.ds(i * _CHUNK, _CHUNK), :], xbuf.at[slot], in_sem.at[slot])

    def copy_out(i, slot):
        return pltpu.make_async_copy(
            ybuf.at[slot], o_hbm.at[pl.ds(i * _CHUNK, _CHUNK), :], out_sem.at[slot])

    for s in range(_NBUF):
        copy_in(s, s).start()

    def step(i, carry):
        slot = jax.lax.rem(i, _NBUF)
        copy_in(i, slot).wait()

        @pl.when(i >= _NBUF)
        def _():
            copy_out(i - _NBUF, slot).wait()

        ybuf[slot] = _gelu(xbuf[slot])
        copy_out(i, slot).start()

        @pl.when(i + _NBUF < n_chunks)
        def _():
            copy_in(i + _NBUF, slot).start()

        return carry

    jax.lax.fori_loop(0, n_chunks, step, 0)

    def drain(i, carry):
        copy_out(i, jax.lax.rem(i, _NBUF)).wait()
        return carry

    jax.lax.fori_loop(n_chunks - _NBUF, n_chunks, drain, 0)


def kernel(x, log_k):
    B, T, D = x.shape
    rows = B * T
    x2 = x.reshape(rows, D)
    n_chunks = rows // _CHUNK
    y2 = pl.pallas_call(
        functools.partial(_body, n_chunks=n_chunks),
        in_specs=[pl.BlockSpec(memory_space=pl.ANY)],
        out_specs=pl.BlockSpec(memory_space=pl.ANY),
        out_shape=jax.ShapeDtypeStruct((rows, D), x.dtype),
        scratch_shapes=[
            pltpu.VMEM((_NBUF, _CHUNK, D), x.dtype),
            pltpu.VMEM((_NBUF, _CHUNK, D), x.dtype),
            pltpu.SemaphoreType.DMA((_NBUF,)),
            pltpu.SemaphoreType.DMA((_NBUF,)),
        ],
    )(x2)
    return y2.reshape(B, T, D)


# speedup vs baseline: 1.0016x; 1.0016x over previous
import functools
import math

import jax
import jax.numpy as jnp
from jax.experimental import pallas as pl
from jax.experimental.pallas import tpu as pltpu

_SQRT_2_OVER_PI = math.sqrt(2.0 / math.pi)

_CHUNK = 512
_NBUF = 6


def _gelu(x):
    inner = _SQRT_2_OVER_PI * (x + 0.044715 * (x * x * x))
    return 0.5 * x * (1.0 + jnp.tanh(inner))


def _body(x_hbm, o_hbm, xbuf, ybuf, in_sem, out_sem, *, n_chunks):
    def copy_in(i, slot):
        return pltpu.make_async_copy(
            x_hbm.at[pl.ds(i * _CHUNK, _CHUNK), :], xbuf.at[slot], in_sem.at[slot])

    def copy_out(i, slot):
        return pltpu.make_async_copy(
            ybuf.at[slot], o_hbm.at[pl.ds(i * _CHUNK, _CHUNK), :], out_sem.at[slot])

    for s in range(_NBUF):
        copy_in(s, s).start()

    def step(i, carry):
        slot = jax.lax.rem(i, _NBUF)
        copy_in(i, slot).wait()

        @pl.when(i >= _NBUF)
        def _():
            copy_out(i - _NBUF, slot).wait()

        ybuf[slot] = _gelu(xbuf[slot])
        copy_out(i, slot).start()

        @pl.when(i + _NBUF < n_chunks)
        def _():
            copy_in(i + _NBUF, slot).start()

        return carry

    jax.lax.fori_loop(0, n_chunks, step, 0)

    def drain(i, carry):
        copy_out(i, jax.lax.rem(i, _NBUF)).wait()
        return carry

    jax.lax.fori_loop(n_chunks - _NBUF, n_chunks, drain, 0)


def kernel(x, log_k):
    B, T, D = x.shape
    rows = B * T
    x2 = x.reshape(rows, D)
    n_chunks = rows // _CHUNK
    y2 = pl.pallas_call(
        functools.partial(_body, n_chunks=n_chunks),
        in_specs=[pl.BlockSpec(memory_space=pl.ANY)],
        out_specs=pl.BlockSpec(memory_space=pl.ANY),
        out_shape=jax.ShapeDtypeStruct((rows, D), x.dtype),
        scratch_shapes=[
            pltpu.VMEM((_NBUF, _CHUNK, D), x.dtype),
            pltpu.VMEM((_NBUF, _CHUNK, D), x.dtype),
            pltpu.SemaphoreType.DMA((_NBUF,)),
            pltpu.SemaphoreType.DMA((_NBUF,)),
        ],
    )(x2)
    return y2.reshape(B, T, D)
